# baseline (speedup 1.0000x reference)
import jax
import jax.numpy as jnp
from jax.experimental import pallas as pl
from jax.experimental.pallas import tpu as pltpu

NORM_PERCENT = 50
CB = 8


def _sumsq_body(x_ref, out_ref):
    xb = x_ref[...]
    out_ref[...] = jnp.sum(xb * xb, axis=1).reshape(1, 1, -1)


def _plan_body(k, s_ref, plan_ref):
    s = s_ref[0, :]
    n = s.shape[0]
    a = jax.lax.broadcast_in_dim(s, (n, n), (0,))
    b = jax.lax.broadcast_in_dim(s, (n, n), (1,))
    jidx = jax.lax.broadcasted_iota(jnp.int32, (n, n), 0)
    cidx = jax.lax.broadcasted_iota(jnp.int32, (n, n), 1)
    beats = (a > b) | ((a == b) & (jidx < cidx))
    rank = jnp.sum(beats.astype(jnp.int32), axis=0)
    plan_ref[0, :] = (rank < k).astype(jnp.int32)


def _mul_body(plan_ref, x_hbm, o_ref, sem):
    b = pl.program_id(0)
    for ch in range(CB):
        c = b * CB + ch

        @pl.when(plan_ref[0, c] == 1)
        def _():
            pltpu.make_async_copy(
                x_hbm.at[pl.ds(c, 1)],
                o_ref.at[pl.ds(ch, 1)],
                sem,
            ).start()

        @pl.when(plan_ref[0, c] == 0)
        def _():
            o_ref[pl.ds(ch, 1), :] = jnp.zeros_like(o_ref[pl.ds(ch, 1), :])

    for ch in range(CB):
        c = b * CB + ch

        @pl.when(plan_ref[0, c] == 1)
        def _():
            pltpu.make_async_copy(
                x_hbm.at[pl.ds(c, 1)],
                o_ref.at[pl.ds(ch, 1)],
                sem,
            ).wait()


def kernel(input):
    x = input
    _, C, H, W = x.shape
    k = int(float(NORM_PERCENT) / 100.0 * float(C))
    if k <= 0 or k >= C:
        k = C
    HW = H * W
    nblk = C // CB

    x2 = x.reshape(C, HW)

    sumsq = pl.pallas_call(
        _sumsq_body,
        grid=(nblk,),
        in_specs=[pl.BlockSpec((CB, HW), lambda i: (i, 0))],
        out_specs=pl.BlockSpec((1, 1, CB), lambda i: (i, 0, 0)),
        out_shape=jax.ShapeDtypeStruct((nblk, 1, CB), jnp.float32),
    )(x2)

    plan = pl.pallas_call(
        lambda s_ref, plan_ref: _plan_body(k, s_ref, plan_ref),
        in_specs=[pl.BlockSpec((1, C), lambda: (0, 0))],
        out_specs=pl.BlockSpec((1, C), lambda: (0, 0)),
        out_shape=jax.ShapeDtypeStruct((1, C), jnp.int32),
    )(sumsq.reshape(1, C))

    grid_spec = pltpu.PrefetchScalarGridSpec(
        num_scalar_prefetch=1,
        grid=(nblk,),
        in_specs=[pl.BlockSpec(memory_space=pl.ANY)],
        out_specs=pl.BlockSpec((CB, HW), lambda i, pref: (i, 0)),
        scratch_shapes=[
            pltpu.SemaphoreType.DMA,
        ],
    )
    out = pl.pallas_call(
        _mul_body,
        grid_spec=grid_spec,
        out_shape=jax.ShapeDtypeStruct((C, HW), jnp.float32),
    )(plan, x2)

    return out.reshape(x.shape)
